# Initial kernel scaffold; baseline (speedup 1.0000x reference)
import jax
import jax.numpy as jnp
from jax.experimental import pallas as pl


def kernel(x, edge_index, e_h, ext_feature, W1, b1, W2, b2):
    raise NotImplementedError("write your pallas kernel here")


# speedup vs baseline: 1.9252x; 1.9252x over previous
import functools

import jax
import jax.numpy as jnp
from jax import lax
from jax.experimental import pallas as pl
from jax.experimental.pallas import tpu as pltpu
from jax.experimental.pallas import tpu_sc as plsc

N = 10000
E = 320000
D = 128
EXT = 16
OUT = 128

_NC = 2
_NS = 16
_NW = _NC * _NS
_EPW = E // _NW
_C = 80
_NCHUNK = _EPW // _C


def _prep_body(x_ref, w1a_ref, w1c_ref, w2a_ref, w1b_ref,
               psrc_ref, pdst_ref, m_ref):
    w2a = w2a_ref[...]
    xa = jnp.dot(x_ref[...], w1a_ref[...], preferred_element_type=jnp.float32)
    psrc_ref[...] = jnp.dot(xa, w2a, preferred_element_type=jnp.float32)
    xc = jnp.dot(x_ref[...], w1c_ref[...], preferred_element_type=jnp.float32)
    pdst_ref[...] = jnp.dot(xc, w2a, preferred_element_type=jnp.float32)

    @pl.when(pl.program_id(0) == 0)
    def _():
        m_ref[...] = jnp.dot(w1b_ref[...], w2a, preferred_element_type=jnp.float32)


def _prep(x, w1a_t, w1c_t, w2a_t, w1b_t):
    bn = 2000
    return pl.pallas_call(
        _prep_body,
        grid=(N // bn,),
        in_specs=[
            pl.BlockSpec((bn, D), lambda i: (i, 0)),
            pl.BlockSpec((D, D), lambda i: (0, 0)),
            pl.BlockSpec((D, D), lambda i: (0, 0)),
            pl.BlockSpec((D, OUT), lambda i: (0, 0)),
            pl.BlockSpec((D, OUT), lambda i: (0, 0)),
        ],
        out_specs=[
            pl.BlockSpec((bn, OUT), lambda i: (i, 0)),
            pl.BlockSpec((bn, OUT), lambda i: (i, 0)),
            pl.BlockSpec((D, OUT), lambda i: (0, 0)),
        ],
        out_shape=[
            jax.ShapeDtypeStruct((N, OUT), jnp.float32),
            jax.ShapeDtypeStruct((N, OUT), jnp.float32),
            jax.ShapeDtypeStruct((D, OUT), jnp.float32),
        ],
    )(x, w1a_t, w1c_t, w2a_t, w1b_t)


def _sc_gather_body(psrc_hbm, pdst_hbm, src_hbm, dst_hbm, g_hbm,
                    idx_s, idx_d, rows_a, rows_b, sem):
    wid = lax.axis_index("s") * _NC + lax.axis_index("c")
    base = wid * _EPW

    def chunk(c, carry):
        off = base + c * _C
        pltpu.sync_copy(src_hbm.at[pl.ds(off, _C)], idx_s)
        pltpu.sync_copy(dst_hbm.at[pl.ds(off, _C)], idx_d)
        cp1 = pltpu.async_copy(psrc_hbm.at[idx_s], rows_a, sem)
        cp2 = pltpu.async_copy(pdst_hbm.at[idx_d], rows_b, sem)
        cp1.wait()
        cp2.wait()

        def row_add(r, carry2):
            for k in range(OUT // 16):
                sl = pl.ds(k * 16, 16)
                rows_a[r, sl] = rows_a[r, sl] + rows_b[r, sl]
            return carry2

        lax.fori_loop(0, _C, row_add, 0, unroll=2)
        pltpu.sync_copy(rows_a, g_hbm.at[pl.ds(off, _C)])
        return carry

    lax.fori_loop(0, _NCHUNK, chunk, 0)


@functools.partial(
    pl.kernel,
    out_type=jax.ShapeDtypeStruct((E, OUT), jnp.float32),
    mesh=plsc.VectorSubcoreMesh(core_axis_name="c", subcore_axis_name="s"),
    scratch_types=[
        pltpu.VMEM((_C,), jnp.int32),
        pltpu.VMEM((_C,), jnp.int32),
        pltpu.VMEM((_C, OUT), jnp.float32),
        pltpu.VMEM((_C, OUT), jnp.float32),
        pltpu.SemaphoreType.DMA,
    ],
)
def _sc_gather(psrc_hbm, pdst_hbm, src_hbm, dst_hbm, g_hbm,
               idx_s, idx_d, rows_a, rows_b, sem):
    _sc_gather_body(psrc_hbm, pdst_hbm, src_hbm, dst_hbm, g_hbm,
                    idx_s, idx_d, rows_a, rows_b, sem)


def _main_body(g_ref, eh_ref, ext_ref, m_ref, wext_ref, w2a_ref,
               b1_ref, b2_ref, out_ref):
    acc = jnp.dot(eh_ref[...], m_ref[...], preferred_element_type=jnp.float32)
    acc = acc + jnp.dot(ext_ref[...], wext_ref[...],
                        preferred_element_type=jnp.float32)
    bias = jnp.dot(b1_ref[...], w2a_ref[...],
                   preferred_element_type=jnp.float32) + b2_ref[...]
    out_ref[...] = jnp.maximum(acc + g_ref[...] + bias, 0.0)


def _main(g, e_h, ext, m, wext_t, w2a_t, b1, b2):
    be = 4000
    return pl.pallas_call(
        _main_body,
        grid=(E // be,),
        in_specs=[
            pl.BlockSpec((be, OUT), lambda i: (i, 0)),
            pl.BlockSpec((be, D), lambda i: (i, 0)),
            pl.BlockSpec((be, EXT), lambda i: (i, 0)),
            pl.BlockSpec((D, OUT), lambda i: (0, 0)),
            pl.BlockSpec((EXT, OUT), lambda i: (0, 0)),
            pl.BlockSpec((D, OUT), lambda i: (0, 0)),
            pl.BlockSpec((1, D), lambda i: (0, 0)),
            pl.BlockSpec((1, OUT), lambda i: (0, 0)),
        ],
        out_specs=pl.BlockSpec((be, OUT), lambda i: (i, 0)),
        out_shape=jax.ShapeDtypeStruct((E, OUT), jnp.float32),
    )(g, e_h, ext, m, wext_t, w2a_t, b1, b2)


def kernel(x, edge_index, e_h, ext_feature, W1, b1, W2, b2):
    w1a_t = W1[:, :D].T
    w1b_t = W1[:, D:2 * D].T
    w1c_t = W1[:, 2 * D:].T
    w2a_t = W2[:, :D].T
    wext_t = W2[:, D:].T
    src = edge_index[0]
    dst = edge_index[1]

    psrc, pdst, m = _prep(x, w1a_t, w1c_t, w2a_t, w1b_t)
    g = _sc_gather(psrc, pdst, src, dst)
    return _main(g, e_h, ext_feature, m, wext_t, w2a_t,
                 b1.reshape(1, D), b2.reshape(1, OUT))


# speedup vs baseline: 3.1209x; 1.6211x over previous
import functools

import jax
import jax.numpy as jnp
from jax import lax
from jax.experimental import pallas as pl
from jax.experimental.pallas import tpu as pltpu
from jax.experimental.pallas import tpu_sc as plsc

N = 10000
E = 320000
D = 128
EXT = 16
OUT = 128

_NC = 2
_NS = 16
_NW = _NC * _NS
_EPW = E // _NW
_C = 40
_NCHUNK = _EPW // _C
_NBUF = 5
_NGRP = _NCHUNK // _NBUF


def _prep_body(x_ref, w1a_ref, w1c_ref, w2a_ref, w1b_ref,
               psrc_ref, pdst_ref, m_ref):
    w2a = w2a_ref[...]
    xa = jnp.dot(x_ref[...], w1a_ref[...], preferred_element_type=jnp.float32)
    psrc_ref[...] = jnp.dot(xa, w2a, preferred_element_type=jnp.float32)
    xc = jnp.dot(x_ref[...], w1c_ref[...], preferred_element_type=jnp.float32)
    pdst_ref[...] = jnp.dot(xc, w2a, preferred_element_type=jnp.float32)

    @pl.when(pl.program_id(0) == 0)
    def _():
        m_ref[...] = jnp.dot(w1b_ref[...], w2a, preferred_element_type=jnp.float32)


def _prep(x, w1a_t, w1c_t, w2a_t, w1b_t):
    bn = 2000
    return pl.pallas_call(
        _prep_body,
        grid=(N // bn,),
        in_specs=[
            pl.BlockSpec((bn, D), lambda i: (i, 0)),
            pl.BlockSpec((D, D), lambda i: (0, 0)),
            pl.BlockSpec((D, D), lambda i: (0, 0)),
            pl.BlockSpec((D, OUT), lambda i: (0, 0)),
            pl.BlockSpec((D, OUT), lambda i: (0, 0)),
        ],
        out_specs=[
            pl.BlockSpec((bn, OUT), lambda i: (i, 0)),
            pl.BlockSpec((bn, OUT), lambda i: (i, 0)),
            pl.BlockSpec((D, OUT), lambda i: (0, 0)),
        ],
        out_shape=[
            jax.ShapeDtypeStruct((N, OUT), jnp.float32),
            jax.ShapeDtypeStruct((N, OUT), jnp.float32),
            jax.ShapeDtypeStruct((D, OUT), jnp.float32),
        ],
    )(x, w1a_t, w1c_t, w2a_t, w1b_t)


def _sc_gather_body(psrc_hbm, pdst_hbm, src_hbm, dst_hbm, g_hbm,
                    idx_s, idx_d, rows_s, rows_d, rows_o, sem_g, sem_wb):
    wid = lax.axis_index("s") * _NC + lax.axis_index("c")
    base = wid * _EPW

    pltpu.sync_copy(src_hbm.at[pl.ds(base, _EPW)], idx_s)
    pltpu.sync_copy(dst_hbm.at[pl.ds(base, _EPW)], idx_d)

    def start_gathers(c, b):
        pltpu.async_copy(psrc_hbm.at[idx_s.at[pl.ds(c * _C, _C)]],
                         rows_s.at[b], sem_g.at[b])
        pltpu.async_copy(pdst_hbm.at[idx_d.at[pl.ds(c * _C, _C)]],
                         rows_d.at[b], sem_g.at[b])

    def wait_gathers(b):
        pltpu.make_async_copy(psrc_hbm.at[pl.ds(0, _C)], rows_s.at[b],
                              sem_g.at[b]).wait()
        pltpu.make_async_copy(psrc_hbm.at[pl.ds(0, _C)], rows_d.at[b],
                              sem_g.at[b]).wait()

    def add(b):
        def row(r, carry):
            for k in range(OUT // 16):
                sl = pl.ds(k * 16, 16)
                rows_o[b, r, sl] = rows_s[b, r, sl] + rows_d[b, r, sl]
            return carry

        lax.fori_loop(0, _C, row, 0, unroll=2)

    def start_wb(c, b):
        pltpu.async_copy(rows_o.at[b], g_hbm.at[pl.ds(base + c * _C, _C)],
                         sem_wb.at[b])

    def wait_wb(b):
        pltpu.make_async_copy(psrc_hbm.at[pl.ds(0, _C)], rows_o.at[b],
                              sem_wb.at[b]).wait()

    for b in range(_NBUF):
        start_gathers(b, b)

    def grp(g, carry):
        for b in range(_NBUF):
            c = g * _NBUF + b
            wait_gathers(b)

            @pl.when(g > 0)
            def _():
                wait_wb(b)

            add(b)
            start_wb(c, b)

            @pl.when(c + _NBUF < _NCHUNK)
            def _():
                start_gathers(c + _NBUF, b)
        return carry

    lax.fori_loop(0, _NGRP, grp, 0)
    for b in range(_NBUF):
        wait_wb(b)


@functools.partial(
    pl.kernel,
    out_type=jax.ShapeDtypeStruct((E, OUT), jnp.float32),
    mesh=plsc.VectorSubcoreMesh(core_axis_name="c", subcore_axis_name="s"),
    scratch_types=[
        pltpu.VMEM((_EPW,), jnp.int32),
        pltpu.VMEM((_EPW,), jnp.int32),
        pltpu.VMEM((_NBUF, _C, OUT), jnp.float32),
        pltpu.VMEM((_NBUF, _C, OUT), jnp.float32),
        pltpu.VMEM((_NBUF, _C, OUT), jnp.float32),
        pltpu.SemaphoreType.DMA((_NBUF,)),
        pltpu.SemaphoreType.DMA((_NBUF,)),
    ],
)
def _sc_gather(psrc_hbm, pdst_hbm, src_hbm, dst_hbm, g_hbm,
               idx_s, idx_d, rows_s, rows_d, rows_o, sem_g, sem_wb):
    _sc_gather_body(psrc_hbm, pdst_hbm, src_hbm, dst_hbm, g_hbm,
                    idx_s, idx_d, rows_s, rows_d, rows_o, sem_g, sem_wb)


def _main_body(g_ref, eh_ref, ext_ref, m_ref, wext_ref, w2a_ref,
               b1_ref, b2_ref, out_ref):
    acc = jnp.dot(eh_ref[...], m_ref[...], preferred_element_type=jnp.float32)
    acc = acc + jnp.dot(ext_ref[...], wext_ref[...],
                        preferred_element_type=jnp.float32)
    bias = jnp.dot(b1_ref[...], w2a_ref[...],
                   preferred_element_type=jnp.float32) + b2_ref[...]
    out_ref[...] = jnp.maximum(acc + g_ref[...] + bias, 0.0)


def _main(g, e_h, ext, m, wext_t, w2a_t, b1, b2):
    be = 4000
    return pl.pallas_call(
        _main_body,
        grid=(E // be,),
        in_specs=[
            pl.BlockSpec((be, OUT), lambda i: (i, 0)),
            pl.BlockSpec((be, D), lambda i: (i, 0)),
            pl.BlockSpec((be, EXT), lambda i: (i, 0)),
            pl.BlockSpec((D, OUT), lambda i: (0, 0)),
            pl.BlockSpec((EXT, OUT), lambda i: (0, 0)),
            pl.BlockSpec((D, OUT), lambda i: (0, 0)),
            pl.BlockSpec((1, D), lambda i: (0, 0)),
            pl.BlockSpec((1, OUT), lambda i: (0, 0)),
        ],
        out_specs=pl.BlockSpec((be, OUT), lambda i: (i, 0)),
        out_shape=jax.ShapeDtypeStruct((E, OUT), jnp.float32),
    )(g, e_h, ext, m, wext_t, w2a_t, b1, b2)


def kernel(x, edge_index, e_h, ext_feature, W1, b1, W2, b2):
    w1a_t = W1[:, :D].T
    w1b_t = W1[:, D:2 * D].T
    w1c_t = W1[:, 2 * D:].T
    w2a_t = W2[:, :D].T
    wext_t = W2[:, D:].T
    src = edge_index[0]
    dst = edge_index[1]

    psrc, pdst, m = _prep(x, w1a_t, w1c_t, w2a_t, w1b_t)
    g = _sc_gather(psrc, pdst, src, dst)
    return _main(g, e_h, ext_feature, m, wext_t, w2a_t,
                 b1.reshape(1, D), b2.reshape(1, OUT))
